# baseline (speedup 1.0000x reference)
import functools

import jax
import jax.numpy as jnp
from jax import lax
from jax.experimental import pallas as pl
from jax.experimental.pallas import tpu as pltpu
from jax.experimental.pallas import tpu_sc as plsc

_NC = 2
_NS = 16
_NW = _NC * _NS
_C = 4


def kernel(x, emb):
    (B,) = x.shape
    V, D = emb.shape
    bpw = B // _NW
    nchunk = bpw // _C

    x2 = x.reshape(_NW, nchunk, _C).astype(jnp.int32)

    mesh = plsc.VectorSubcoreMesh(core_axis_name="c", subcore_axis_name="s")

    @functools.partial(
        pl.kernel,
        out_type=jax.ShapeDtypeStruct((B, D), emb.dtype),
        mesh=mesh,
        scratch_types=[
            pltpu.VMEM((nchunk, _C), jnp.int32),
            pltpu.VMEM((_C, D), emb.dtype),
            pltpu.VMEM((_C, D), emb.dtype),
            pltpu.SemaphoreType.DMA,
            pltpu.SemaphoreType.DMA,
            pltpu.SemaphoreType.DMA,
            pltpu.SemaphoreType.DMA,
        ],
    )
    def gather_k(x_hbm, emb_hbm, out_hbm, idx_v, b0, b1, sg0, sg1, sw0, sw1):
        wid = lax.axis_index("s") * _NC + lax.axis_index("c")
        rbase = wid * bpw
        pltpu.sync_copy(x_hbm.at[wid], idx_v)

        pltpu.async_copy(emb_hbm.at[idx_v.at[0]], b0, sg0)

        @pl.loop(0, nchunk, step=2)
        def _(j):
            @pl.when(j > 0)
            def _():
                pltpu.make_async_copy(
                    b1, out_hbm.at[pl.ds(rbase + (j - 1) * _C, _C)], sw1
                ).wait()

            pltpu.async_copy(emb_hbm.at[idx_v.at[j + 1]], b1, sg1)
            pltpu.make_async_copy(emb_hbm.at[idx_v.at[j]], b0, sg0).wait()
            pltpu.async_copy(b0, out_hbm.at[pl.ds(rbase + j * _C, _C)], sw0)
            pltpu.make_async_copy(emb_hbm.at[idx_v.at[j + 1]], b1, sg1).wait()
            pltpu.async_copy(b1, out_hbm.at[pl.ds(rbase + (j + 1) * _C, _C)], sw1)
            pltpu.make_async_copy(
                b0, out_hbm.at[pl.ds(rbase + j * _C, _C)], sw0
            ).wait()

            @pl.when(j + 2 < nchunk)
            def _():
                pltpu.async_copy(emb_hbm.at[idx_v.at[j + 2]], b0, sg0)

        pltpu.make_async_copy(
            b1, out_hbm.at[pl.ds(rbase + (nchunk - 1) * _C, _C)], sw1
        ).wait()

    return gather_k(x2, emb)
